# Initial kernel scaffold; baseline (speedup 1.0000x reference)
import jax
import jax.numpy as jnp
from jax.experimental import pallas as pl


def kernel(x, edge_index, batch, W1_0, W1_1, b1, W2_0, W2_1, b2, gamma, beta, linW, linb):
    raise NotImplementedError("write your pallas kernel here")


# speedup vs baseline: 11.8731x; 11.8731x over previous
import functools

import jax
import jax.numpy as jnp
from jax import lax
from jax.experimental import pallas as pl
from jax.experimental.pallas import tpu as pltpu
from jax.experimental.pallas import tpu_sc as plsc

N = 10000
E = 320000
D = 128
H = 128
G = 64

NC = 2
NS = 16
NW = NC * NS
LN = 128
CHUNKS = -(-E // (NW * LN))
EPT = CHUNKS * LN
EPAD = NW * EPT
DEG_N = 10240
DUMMY = N
ROWS_Z = DEG_N // NS

def _sc_prep_body(src3, dst3, zeros1, deg_parts, dstp3, src_v, dst_v, dstp_v,
                  deg_v):
    c = lax.axis_index("c")
    s = lax.axis_index("s")
    wid = s * NC + c
    pltpu.sync_copy(src3.at[wid], src_v)
    pltpu.sync_copy(dst3.at[wid], dst_v)
    pltpu.sync_copy(zeros1, deg_v)
    ones = jnp.full((16,), 1.0, dtype=jnp.float32)

    def body(j, _):
        for k in range(LN // 16):
            sv = src_v[j, pl.ds(k * 16, 16)]
            dv = dst_v[j, pl.ds(k * 16, 16)]
            bad = (sv == dv) | (dv >= N)
            sp = jnp.where(bad, DUMMY, sv)
            dp = jnp.where(bad, DUMMY, dv)
            dstp_v[j, pl.ds(k * 16, 16)] = dp
            plsc.addupdate_scatter(deg_v, [sp], ones)
        return 0

    lax.fori_loop(0, CHUNKS, body, 0)
    pltpu.sync_copy(deg_v, deg_parts.at[wid])
    pltpu.sync_copy(dstp_v, dstp3.at[wid])


@functools.cache
def _sc_prep():
    mesh = plsc.VectorSubcoreMesh(core_axis_name="c", subcore_axis_name="s",
                                  num_cores=NC, num_subcores=NS)
    return pl.kernel(
        _sc_prep_body,
        out_type=[
            jax.ShapeDtypeStruct((NW, DEG_N), jnp.float32),
            jax.ShapeDtypeStruct((NW, CHUNKS, LN), jnp.int32),
        ],
        mesh=mesh,
        compiler_params=pltpu.CompilerParams(needs_layout_passes=False),
        scratch_types=[
            pltpu.VMEM((CHUNKS, LN), jnp.int32),
            pltpu.VMEM((CHUNKS, LN), jnp.int32),
            pltpu.VMEM((CHUNKS, LN), jnp.int32),
            pltpu.VMEM((DEG_N,), jnp.float32),
        ],
    )


def _sc_conv_body(y, src3, dstp3, zeros2, parts, src_v, dstp_v, rows_v, sem,
                  acc_sh):
    c = lax.axis_index("c")
    s = lax.axis_index("s")
    wid = s * NC + c
    pltpu.sync_copy(src3.at[wid], src_v)
    pltpu.sync_copy(dstp3.at[wid], dstp_v)
    pltpu.sync_copy(zeros2.at[pl.ds(s * ROWS_Z, ROWS_Z)],
                    acc_sh.at[pl.ds(s * ROWS_Z, ROWS_Z)])
    plsc.subcore_barrier()

    def body(j, _):
        pltpu.async_copy(y.at[src_v.at[j]], rows_v, sem).wait()
        pltpu.sync_copy(rows_v, acc_sh.at[dstp_v.at[j]], add=True)
        return 0

    lax.fori_loop(0, CHUNKS, body, 0)
    plsc.subcore_barrier()
    pltpu.sync_copy(acc_sh.at[pl.ds(s * ROWS_Z, ROWS_Z)],
                    parts.at[c, pl.ds(s * ROWS_Z, ROWS_Z)])


@functools.cache
def _sc_conv():
    mesh = plsc.VectorSubcoreMesh(core_axis_name="c", subcore_axis_name="s",
                                  num_cores=NC, num_subcores=NS)
    return pl.kernel(
        _sc_conv_body,
        out_type=jax.ShapeDtypeStruct((NC, DEG_N, H), jnp.float32),
        mesh=mesh,
        compiler_params=pltpu.CompilerParams(needs_layout_passes=False),
        scratch_types=[
            pltpu.VMEM((CHUNKS, LN), jnp.int32),
            pltpu.VMEM((CHUNKS, LN), jnp.int32),
            pltpu.VMEM((LN, H), jnp.float32),
            pltpu.SemaphoreType.DMA,
            pltpu.VMEM_SHARED((DEG_N, H), jnp.float32),
        ],
    )


def _tc_pre_body(deg_ref, x_ref, w1_ref, w0_ref, b1_ref, dis_ref, y1_ref,
                 xw0_ref):
    deg = jnp.sum(deg_ref[...], axis=0)[:N, None]
    dis = jnp.where(deg > 0, 1.0 / jnp.sqrt(deg), 0.0)
    dis_ref[...] = dis
    x = x_ref[...]
    y1_ref[...] = dis * jnp.dot(x, w1_ref[...],
                                preferred_element_type=jnp.float32)
    xw0_ref[...] = jnp.dot(x, w0_ref[...],
                           preferred_element_type=jnp.float32) + b1_ref[...]


def _tc_mid_body(parts_ref, dis_ref, xw0_ref, gamma_ref, beta_ref, w20_ref,
                 w21_ref, b2_ref, y2_ref, hw0_ref):
    dis = dis_ref[...]
    h = xw0_ref[...] - dis * (parts_ref[0, :N] + parts_ref[1, :N])
    mean = jnp.mean(h, axis=0, keepdims=True)
    var = jnp.mean((h - mean) ** 2, axis=0, keepdims=True)
    h = (h - mean) / jnp.sqrt(var + 1e-5)
    h = gamma_ref[...] * h + beta_ref[...]
    h = jnp.where(h > 0, h, 0.01 * h)
    y2_ref[...] = dis * jnp.dot(h, w21_ref[...],
                                preferred_element_type=jnp.float32)
    hw0_ref[...] = jnp.dot(h, w20_ref[...],
                           preferred_element_type=jnp.float32) + b2_ref[...]


def _tc_post_body(parts_ref, dis_ref, hw0_ref, batch_ref, linw_ref, linb_ref,
                  out_ref):
    h2 = hw0_ref[...] - dis_ref[...] * (parts_ref[0, :N] + parts_ref[1, :N])
    gids = lax.broadcasted_iota(jnp.int32, (1, G), 1)
    oh = (batch_ref[...] == gids).astype(jnp.float32)
    sums = lax.dot_general(oh, h2, (((0,), (0,)), ((), ())),
                           preferred_element_type=jnp.float32)
    cnt = jnp.sum(oh, axis=0)[:, None]
    pooled = sums / jnp.maximum(cnt, 1.0)
    out_ref[...] = jnp.dot(pooled, linw_ref[...],
                           preferred_element_type=jnp.float32) + linb_ref[...]


@jax.jit
def kernel(x, edge_index, batch, W1_0, W1_1, b1, W2_0, W2_1, b2, gamma, beta,
           linW, linb):
    src = edge_index[0]
    dst = edge_index[1]
    pad = EPAD - E
    src3 = jnp.concatenate([src, jnp.zeros((pad,), jnp.int32)]).reshape(
        NW, CHUNKS, LN)
    dst3 = jnp.concatenate([dst, jnp.full((pad,), N, jnp.int32)]).reshape(
        NW, CHUNKS, LN)
    zeros1 = jnp.zeros((DEG_N,), jnp.float32)
    zeros2 = jnp.zeros((DEG_N, H), jnp.float32)

    deg_parts, dstp3 = _sc_prep()(src3, dst3, zeros1)

    dis, y1, xw0 = pl.pallas_call(
        _tc_pre_body,
        out_shape=[
            jax.ShapeDtypeStruct((N, 1), jnp.float32),
            jax.ShapeDtypeStruct((N, H), jnp.float32),
            jax.ShapeDtypeStruct((N, H), jnp.float32),
        ],
    )(deg_parts, x, W1_1, W1_0, b1.reshape(1, H))

    parts1 = _sc_conv()(y1, src3, dstp3, zeros2)

    y2, hw0 = pl.pallas_call(
        _tc_mid_body,
        out_shape=[
            jax.ShapeDtypeStruct((N, H), jnp.float32),
            jax.ShapeDtypeStruct((N, H), jnp.float32),
        ],
    )(parts1, dis, xw0, gamma.reshape(1, H), beta.reshape(1, H), W2_0, W2_1,
      b2.reshape(1, H))

    parts2 = _sc_conv()(y2, src3, dstp3, zeros2)

    out = pl.pallas_call(
        _tc_post_body,
        out_shape=jax.ShapeDtypeStruct((G, 1), jnp.float32),
    )(parts2, dis, hw0, batch.reshape(N, 1), linW, linb.reshape(1, 1))
    return out
